# Initial kernel scaffold; baseline (speedup 1.0000x reference)
import jax
import jax.numpy as jnp
from jax.experimental import pallas as pl


def kernel(pred, A, org_ind, bound_num):
    raise NotImplementedError("write your pallas kernel here")


# speedup vs baseline: 4.6321x; 4.6321x over previous
import functools

import jax
import jax.numpy as jnp
from jax import lax
from jax.experimental import pallas as pl
from jax.experimental.pallas import tpu as pltpu
from jax.experimental.pallas import tpu_sc as plsc

_L = 16


@functools.lru_cache(maxsize=None)
def _make_sc_kernel(B, Nv, K, SB):
    info = plsc.get_sparse_core_info()
    NC, NS = info.num_cores, info.num_subcores
    NW = NC * NS
    QP = NW // B
    assert Nv % QP == 0
    chunk = Nv // QP
    n_full = chunk // SB
    tail = chunk - n_full * SB
    tb = tail // _L
    rem = tail - tb * _L

    mesh = plsc.VectorSubcoreMesh(core_axis_name="c", subcore_axis_name="s")

    @functools.partial(
        pl.kernel,
        out_type=jax.ShapeDtypeStruct((NW, _L), jnp.float32),
        mesh=mesh,
        compiler_params=pltpu.CompilerParams(needs_layout_passes=False),
        scratch_types=[
            pltpu.VMEM((Nv,), jnp.float32),
            pltpu.VMEM((Nv,), jnp.float32),
            pltpu.VMEM((SB * K,), jnp.int32),
            pltpu.VMEM((SB * K,), jnp.float32),
            pltpu.VMEM((_L,), jnp.int32),
            pltpu.VMEM((_L,), jnp.float32),
        ],
    )
    def sc(pred_hbm, a_hbm, ind_hbm, bidx_hbm, out_hbm,
           x_v, y_v, ind_v, a_v, bidx_v, acc_v):
        wid = lax.axis_index("s") * NC + lax.axis_index("c")
        b = wid // QP
        q = wid - b * QP
        cstart = q * chunk
        iota = lax.iota(jnp.int32, _L)

        pltpu.sync_copy(pred_hbm.at[b, 0], x_v)
        pltpu.sync_copy(pred_hbm.at[b, 1], y_v)
        pltpu.sync_copy(bidx_hbm, bidx_v)
        bvec = bidx_v[...]

        def do_block(rows, n0, lmask):
            fbase = rows * K
            sx = jnp.zeros((_L,), jnp.float32)
            sy = jnp.zeros((_L,), jnp.float32)
            for k in range(K):
                fidx = fbase + k
                if lmask is None:
                    icol = plsc.load_gather(ind_v, [fidx])
                    acol = plsc.load_gather(a_v, [fidx])
                else:
                    icol = plsc.load_gather(ind_v, [fidx], mask=lmask)
                    icol = jnp.where(lmask, icol, 0)
                    acol = plsc.load_gather(a_v, [fidx], mask=lmask)
                    acol = jnp.where(lmask, acol, jnp.float32(0.0))
                xv = plsc.load_gather(x_v, [icol])
                yv = plsc.load_gather(y_v, [icol])
                sx = sx + acol * xv
                sy = sy + acol * yv
            nid = n0 + rows
            m = (nid != 0) & (nid != bvec)
            if lmask is not None:
                m = m & lmask
            return jnp.where(m, jnp.abs(sx) + jnp.abs(sy), jnp.float32(0.0))

        def run_blocks(n0, nblocks, acc):
            def jbody(j, a):
                rows = j * _L + iota
                return a + do_block(rows, n0, None)
            return lax.fori_loop(0, nblocks, jbody, acc)

        acc = jnp.zeros((_L,), jnp.float32)

        def sb_body(sb, a):
            n0 = cstart + sb * SB
            pltpu.sync_copy(ind_hbm.at[pl.ds(n0 * K, SB * K)], ind_v)
            pltpu.sync_copy(a_hbm.at[b, pl.ds(n0 * K, SB * K)], a_v)
            return run_blocks(n0, SB // _L, a)

        acc = lax.fori_loop(0, n_full, sb_body, acc)

        if tail:
            n0 = cstart + n_full * SB
            pltpu.sync_copy(ind_hbm.at[pl.ds(n0 * K, tail * K)],
                            ind_v.at[pl.ds(0, tail * K)])
            pltpu.sync_copy(a_hbm.at[b, pl.ds(n0 * K, tail * K)],
                            a_v.at[pl.ds(0, tail * K)])
            if tb:
                acc = run_blocks(n0, tb, acc)
            if rem:
                rows = tb * _L + iota
                acc = acc + do_block(rows, n0, rows < tail)

        acc_v[...] = acc
        pltpu.sync_copy(acc_v, out_hbm.at[wid])

    return sc


def kernel(pred, A, org_ind, bound_num):
    B, _, Nv = pred.shape
    K = A.shape[2]
    bidx = jnp.floor(bound_num / 2).astype(jnp.int32)
    bidx_arr = jnp.full((_L,), bidx, jnp.int32)
    a2 = A.reshape(B, Nv * K)
    ind2 = jnp.asarray(org_ind, jnp.int32).reshape(Nv * K)
    partials = _make_sc_kernel(B, Nv, K, 256)(pred, a2, ind2, bidx_arr)
    return jnp.sum(partials) / jnp.float32(2 * B * (Nv - 1))


# speedup vs baseline: 12.6061x; 2.7215x over previous
import functools

import jax
import jax.numpy as jnp
from jax import lax
from jax.experimental import pallas as pl
from jax.experimental.pallas import tpu as pltpu
from jax.experimental.pallas import tpu_sc as plsc

_L = 16


@functools.lru_cache(maxsize=None)
def _make_sc_kernel(B, Nv, K, SB):
    info = plsc.get_sparse_core_info()
    NC, NS = info.num_cores, info.num_subcores
    NW = NC * NS
    QP = NW // B
    full_sbs = Nv // SB
    sb_extra = full_sbs % QP
    sb_base = full_sbs // QP
    tail = Nv - full_sbs * SB
    tail_n0 = full_sbs * SB
    tb = tail // _L
    rem = tail - tb * _L

    mesh = plsc.VectorSubcoreMesh(core_axis_name="c", subcore_axis_name="s")

    @functools.partial(
        pl.kernel,
        out_type=jax.ShapeDtypeStruct((NW, _L), jnp.float32),
        mesh=mesh,
        compiler_params=pltpu.CompilerParams(
            needs_layout_passes=False, use_tc_tiling_on_sc=False),
        scratch_types=[
            pltpu.VMEM((Nv,), jnp.float32),
            pltpu.VMEM((Nv,), jnp.float32),
            pltpu.VMEM((SB, K), jnp.int32),
            pltpu.VMEM((SB, K), jnp.float32),
            pltpu.VMEM((_L,), jnp.int32),
            pltpu.VMEM((_L,), jnp.float32),
        ],
    )
    def sc(pred_hbm, a_hbm, ind_hbm, bidx_hbm, out_hbm,
           x_v, y_v, ind_v, a_v, bidx_v, acc_v):
        wid = lax.axis_index("s") * NC + lax.axis_index("c")
        b = wid // QP
        q = wid - b * QP
        iota = lax.iota(jnp.int32, _L)

        pltpu.sync_copy(pred_hbm.at[b, 0], x_v)
        pltpu.sync_copy(pred_hbm.at[b, 1], y_v)
        pltpu.sync_copy(bidx_hbm, bidx_v)
        bvec = bidx_v[...]

        def do_block(rows, n0, lmask):
            zero = jnp.zeros((_L,), jnp.float32)
            sx = [zero, zero]
            sy = [zero, zero]
            for k in range(K):
                colk = jnp.full((_L,), k, jnp.int32)
                if lmask is None:
                    icol = plsc.load_gather(ind_v, [rows, colk])
                    acol = plsc.load_gather(a_v, [rows, colk])
                else:
                    icol = plsc.load_gather(ind_v, [rows, colk], mask=lmask)
                    icol = jnp.where(lmask, icol, 0)
                    acol = plsc.load_gather(a_v, [rows, colk], mask=lmask)
                    acol = jnp.where(lmask, acol, jnp.float32(0.0))
                xv = plsc.load_gather(x_v, [icol])
                yv = plsc.load_gather(y_v, [icol])
                sx[k & 1] = sx[k & 1] + acol * xv
                sy[k & 1] = sy[k & 1] + acol * yv
            nid = n0 + rows
            m = (nid != 0) & (nid != bvec)
            if lmask is not None:
                m = m & lmask
            mag = jnp.abs(sx[0] + sx[1]) + jnp.abs(sy[0] + sy[1])
            return jnp.where(m, mag, jnp.float32(0.0))

        def run_blocks(n0, nblocks, acc):
            def jbody(j, a):
                rows = j * _L + iota
                return a + do_block(rows, n0, None)
            return lax.fori_loop(0, nblocks, jbody, acc)

        acc = jnp.zeros((_L,), jnp.float32)

        def sb_body(i, a):
            n0 = (q + i * QP) * SB
            pltpu.sync_copy(ind_hbm.at[pl.ds(n0, SB)], ind_v)
            pltpu.sync_copy(a_hbm.at[b, pl.ds(n0, SB)], a_v)
            return run_blocks(n0, SB // _L, a)

        nsb = jnp.where(q < sb_extra, sb_base + 1, sb_base)
        acc = lax.fori_loop(0, nsb, sb_body, acc)

        if tail:
            tail_q = QP - 1 if sb_extra else 0

            @pl.when(q == tail_q)
            def _():
                t = jnp.zeros((_L,), jnp.float32)
                pltpu.sync_copy(ind_hbm.at[pl.ds(tail_n0, tail)],
                                ind_v.at[pl.ds(0, tail)])
                pltpu.sync_copy(a_hbm.at[b, pl.ds(tail_n0, tail)],
                                a_v.at[pl.ds(0, tail)])
                if tb:
                    t = run_blocks(tail_n0, tb, t)
                if rem:
                    rows = tb * _L + iota
                    t = t + do_block(rows, tail_n0, rows < tail)
                acc_v[...] = t

            @pl.when(q != tail_q)
            def _():
                acc_v[...] = jnp.zeros((_L,), jnp.float32)

            acc_v[...] = acc_v[...] + acc
        else:
            acc_v[...] = acc
        pltpu.sync_copy(acc_v, out_hbm.at[wid])

    return sc


def kernel(pred, A, org_ind, bound_num):
    B, _, Nv = pred.shape
    K = A.shape[2]
    bidx = jnp.floor(bound_num / 2).astype(jnp.int32)
    bidx_arr = jnp.full((_L,), bidx, jnp.int32)
    ind = jnp.asarray(org_ind, jnp.int32)
    partials = _make_sc_kernel(B, Nv, K, 256)(pred, A, ind, bidx_arr)
    return jnp.sum(partials) / jnp.float32(2 * B * (Nv - 1))
